# baseline (speedup 1.0000x reference)
import jax
import jax.numpy as jnp
from jax import lax
from jax.experimental import pallas as pl
from jax.experimental.pallas import tpu as pltpu
from jax.experimental.pallas import tpu_sc as plsc

N = 10000
E = 320000
D = 128
H = 64
R = 8
OUT = H // 2

NS = 16
NC = 2
EPR = E // R
EPT = EPR // NS
CH = 128
NCHUNK = EPT // 125
CPC = EPT // NCHUNK
TROWS = 624
TAIL = N - NS * TROWS
ACC_ROWS = N + 8
W = 128


_BN = 1000


def _tc_mid_body(a_ref, c_ref, w0_ref, b0_ref, w1_ref, out_ref):
    hs = []
    for r in range(R):
        z = jnp.dot(a_ref[r], w0_ref[r], preferred_element_type=jnp.float32)
        recip = 1.0 / jnp.maximum(c_ref[r][:, 0:1], 1.0)
        hs.append(z * recip)
    h = jnp.concatenate(hs, axis=1) + b0_ref[...]
    h = jnp.maximum(h, 0.0)
    for p in range(R // 2):
        out_ref[p] = jnp.dot(h, w1_ref[p], preferred_element_type=jnp.float32)


def _tc_mid(A0, cnt, W0, b0f, W1p):
    return pl.pallas_call(
        _tc_mid_body,
        grid=(N // _BN,),
        in_specs=[
            pl.BlockSpec((R, _BN, D), lambda i: (0, i, 0)),
            pl.BlockSpec((R, _BN, W), lambda i: (0, i, 0)),
            pl.BlockSpec((R, D, H), lambda i: (0, 0, 0)),
            pl.BlockSpec((1, R * H), lambda i: (0, 0)),
            pl.BlockSpec((R // 2, R * H, 2 * H), lambda i: (0, 0, 0)),
        ],
        out_specs=pl.BlockSpec((R // 2, _BN, 2 * H), lambda i: (0, i, 0)),
        out_shape=jax.ShapeDtypeStruct((R // 2, N, 2 * H), jnp.float32),
    )(A0, cnt, W0, b0f, W1p)


def _tc_head_body(a_ref, c_ref, b1_ref, wl_ref, bl_ref, out_ref):
    s = None
    for r in range(R):
        off = (r % 2) * H
        recip = 1.0 / jnp.maximum(c_ref[r][:, 0:1], 1.0)
        t = a_ref[r][:, off:off + H] * recip
        s = t if s is None else s + t
    pre = s * (1.0 / R) + jnp.mean(b1_ref[...], axis=0, keepdims=True)
    logits = jnp.dot(pre, wl_ref[...], preferred_element_type=jnp.float32)
    logits = logits + bl_ref[...]
    m = jnp.max(logits, axis=1, keepdims=True)
    lse = jnp.log(jnp.sum(jnp.exp(logits - m), axis=1, keepdims=True)) + m
    out_ref[...] = logits - lse


def _tc_head(A1, cnt, b1, Wl, blf):
    return pl.pallas_call(
        _tc_head_body,
        grid=(N // _BN,),
        in_specs=[
            pl.BlockSpec((R, _BN, W), lambda i: (0, i, 0)),
            pl.BlockSpec((R, _BN, W), lambda i: (0, i, 0)),
            pl.BlockSpec((R, H), lambda i: (0, 0)),
            pl.BlockSpec((H, OUT), lambda i: (0, 0)),
            pl.BlockSpec((1, OUT), lambda i: (0, 0)),
        ],
        out_specs=pl.BlockSpec((_BN, OUT), lambda i: (i, 0)),
        out_shape=jax.ShapeDtypeStruct((N, OUT), jnp.float32),
    )(A1, cnt, b1, Wl, blf)


def _sc_cnt_body(didx_hbm, z_hbm, o_hbm, out_hbm, acc, gdst, buf0, buf1):
    c = lax.axis_index("c")
    s = lax.axis_index("s")
    pltpu.sync_copy(o_hbm, buf1)

    for r_loc in range(R // NC):
        r = c * (R // NC) + r_loc
        pltpu.sync_copy(didx_hbm.at[r, s], gdst)

        pltpu.sync_copy(z_hbm, buf0)
        for k in range(5):
            row0 = s * TROWS + k * CH
            nrows = min(CH, TROWS - k * CH)
            pltpu.sync_copy(buf0.at[pl.ds(0, nrows)],
                            acc.at[pl.ds(row0, nrows)])
        @pl.when(s == NS - 1)
        def _():
            pltpu.sync_copy(buf0.at[pl.ds(0, TAIL + 8)],
                            acc.at[pl.ds(NS * TROWS, TAIL + 8)])

        plsc.subcore_barrier()

        @pl.loop(0, NCHUNK)
        def _scatter(j):
            pltpu.sync_copy(buf1, acc.at[gdst.at[j]], add=True)

        plsc.subcore_barrier()

        for k in range(5):
            row0 = s * TROWS + k * CH
            nrows = min(CH, TROWS - k * CH)
            pltpu.sync_copy(acc.at[pl.ds(row0, nrows)],
                            out_hbm.at[pl.ds(r * N + row0, nrows)])
        @pl.when(s == NS - 1)
        def _():
            pltpu.sync_copy(acc.at[pl.ds(NS * TROWS, TAIL)],
                            out_hbm.at[pl.ds(r * N + NS * TROWS, TAIL)])

        plsc.subcore_barrier()


def _sc_cnt(didx):
    mesh = plsc.VectorSubcoreMesh(core_axis_name="c", subcore_axis_name="s",
                                  num_cores=NC, num_subcores=NS)
    kern = pl.kernel(
        _sc_cnt_body,
        out_type=jax.ShapeDtypeStruct((R * N, W), jnp.float32),
        mesh=mesh,
        scratch_types=[
            pltpu.VMEM_SHARED((ACC_ROWS, W), jnp.float32),
            pltpu.VMEM((NCHUNK, CH), jnp.int32),
            pltpu.VMEM((CH, W), jnp.float32),
            pltpu.VMEM((CH, W), jnp.float32),
        ],
    )
    return kern(didx, jnp.zeros((CH, W), jnp.float32),
                jnp.ones((CH, W), jnp.float32))


def _sc_agg_body(t_hbm, gidx_hbm, didx_hbm, z_hbm, out_hbm,
                 acc, gsrc, gdst, buf0, buf1, sem0, sem1):
    c = lax.axis_index("c")
    s = lax.axis_index("s")

    for r_loc in range(R // NC):
        r = c * (R // NC) + r_loc

        pltpu.sync_copy(gidx_hbm.at[r, s], gsrc)
        pltpu.sync_copy(didx_hbm.at[r, s], gdst)

        pltpu.sync_copy(z_hbm, buf0)
        for k in range(5):
            row0 = s * TROWS + k * CH
            nrows = min(CH, TROWS - k * CH)
            pltpu.sync_copy(buf0.at[pl.ds(0, nrows)],
                            acc.at[pl.ds(row0, nrows)])
        @pl.when(s == NS - 1)
        def _():
            pltpu.sync_copy(buf0.at[pl.ds(0, TAIL + 8)],
                            acc.at[pl.ds(NS * TROWS, TAIL + 8)])

        pltpu.async_copy(t_hbm.at[gsrc.at[0]], buf0, sem0)
        pltpu.async_copy(t_hbm.at[gsrc.at[1]], buf1, sem1)

        plsc.subcore_barrier()

        @pl.loop(0, NCHUNK - 2, step=2)
        def _scatter(j):
            for b, (buf, sem) in enumerate(((buf0, sem0), (buf1, sem1))):
                jj = j + b
                pltpu.make_async_copy(t_hbm.at[gsrc.at[jj]], buf, sem).wait()
                pltpu.sync_copy(buf, acc.at[gdst.at[jj]], add=True)
                pltpu.async_copy(t_hbm.at[gsrc.at[jj + 2]], buf, sem)

        for b, (buf, sem) in enumerate(((buf0, sem0), (buf1, sem1))):
            jj = NCHUNK - 2 + b
            pltpu.make_async_copy(t_hbm.at[gsrc.at[jj]], buf, sem).wait()
            pltpu.sync_copy(buf, acc.at[gdst.at[jj]], add=True)

        plsc.subcore_barrier()

        for k in range(5):
            row0 = s * TROWS + k * CH
            nrows = min(CH, TROWS - k * CH)
            pltpu.sync_copy(acc.at[pl.ds(row0, nrows)],
                            out_hbm.at[pl.ds(r * N + row0, nrows)])
        @pl.when(s == NS - 1)
        def _():
            pltpu.sync_copy(acc.at[pl.ds(NS * TROWS, TAIL)],
                            out_hbm.at[pl.ds(r * N + NS * TROWS, TAIL)])

        plsc.subcore_barrier()


def _sc_agg(T, gidx, didx):
    mesh = plsc.VectorSubcoreMesh(core_axis_name="c", subcore_axis_name="s",
                                  num_cores=NC, num_subcores=NS)
    scratch = [
        pltpu.VMEM_SHARED((ACC_ROWS, W), jnp.float32),
        pltpu.VMEM((NCHUNK, CH), jnp.int32),
        pltpu.VMEM((NCHUNK, CH), jnp.int32),
        pltpu.VMEM((CH, W), jnp.float32),
        pltpu.VMEM((CH, W), jnp.float32),
        pltpu.SemaphoreType.DMA,
        pltpu.SemaphoreType.DMA,
    ]
    kern = pl.kernel(
        _sc_agg_body,
        out_type=jax.ShapeDtypeStruct((R * N, W), jnp.float32),
        mesh=mesh,
        scratch_types=scratch,
    )
    return kern(T, gidx, didx, jnp.zeros((CH, W), jnp.float32))


def kernel(x, edge_index, W0, b0, W1, b1, Wl, bl):
    src = edge_index[0]
    dst = edge_index[1]

    def _prep(ix, pad_val):
        a = ix.reshape(EPR, R).T.reshape(R, NS, NCHUNK, CPC)
        return jnp.pad(a, ((0, 0), (0, 0), (0, 0), (0, CH - CPC)),
                       constant_values=pad_val)

    srcp = _prep(src, 0)
    didx_p = _prep(dst, N)
    spread = (jnp.arange(E, dtype=jnp.int32) // R) % 4 * N
    gidx0 = _prep(src + spread, 0)
    pair_off = (jnp.arange(R, dtype=jnp.int32) // 2 * N).reshape(R, 1, 1, 1)
    gidx1 = srcp + pair_off

    W1p = W1.reshape(R // 2, 2, R * H, H).transpose(0, 2, 1, 3)
    W1p = W1p.reshape(R // 2, R * H, 2 * H)

    cnt = _sc_cnt(didx_p).reshape(R, N, W)
    xdup = jnp.concatenate([x, x, x, x], axis=0)
    A0 = _sc_agg(xdup, gidx0, didx_p)
    U = _tc_mid(A0.reshape(R, N, D), cnt, W0, b0.reshape(1, R * H), W1p)
    A1 = _sc_agg(U.reshape(R // 2 * N, 2 * H), gidx1, didx_p)
    return _tc_head(A1.reshape(R, N, W), cnt, b1, Wl, bl.reshape(1, OUT))


# speedup vs baseline: 1.0387x; 1.0387x over previous
import jax
import jax.numpy as jnp
from jax import lax
from jax.experimental import pallas as pl
from jax.experimental.pallas import tpu as pltpu
from jax.experimental.pallas import tpu_sc as plsc

N = 10000
E = 320000
D = 128
H = 64
R = 8
OUT = H // 2

NS = 16
NC = 2
EPR = E // R
EPT = EPR // NS
CH = 128
NCHUNK = EPT // 125
CPC = EPT // NCHUNK
TROWS = 624
TAIL = N - NS * TROWS
ACC_ROWS = N + 8
W = 128


_BN = 1000


def _tc_mid_body(a_ref, c_ref, w0_ref, b0_ref, w1_ref, out_ref):
    hs = []
    for r in range(R):
        z = jnp.dot(a_ref[r], w0_ref[r], preferred_element_type=jnp.float32)
        recip = 1.0 / jnp.maximum(c_ref[r][:, 0:1], 1.0)
        hs.append(z * recip)
    h = jnp.concatenate(hs, axis=1) + b0_ref[...]
    h = jnp.maximum(h, 0.0)
    for p in range(R // 2):
        out_ref[p] = jnp.dot(h, w1_ref[p], preferred_element_type=jnp.float32)


def _tc_mid(A0, cnt, W0, b0f, W1p):
    return pl.pallas_call(
        _tc_mid_body,
        grid=(N // _BN,),
        in_specs=[
            pl.BlockSpec((R, _BN, D), lambda i: (0, i, 0)),
            pl.BlockSpec((R, _BN, W), lambda i: (0, i, 0)),
            pl.BlockSpec((R, D, H), lambda i: (0, 0, 0)),
            pl.BlockSpec((1, R * H), lambda i: (0, 0)),
            pl.BlockSpec((R // 2, R * H, 2 * H), lambda i: (0, 0, 0)),
        ],
        out_specs=pl.BlockSpec((R // 2, _BN, 2 * H), lambda i: (0, i, 0)),
        out_shape=jax.ShapeDtypeStruct((R // 2, N, 2 * H), jnp.float32),
    )(A0, cnt, W0, b0f, W1p)


def _tc_head_body(a_ref, c_ref, b1_ref, wl_ref, bl_ref, out_ref):
    s = None
    for r in range(R):
        off = (r % 2) * H
        recip = 1.0 / jnp.maximum(c_ref[r][:, 0:1], 1.0)
        t = a_ref[r][:, off:off + H] * recip
        s = t if s is None else s + t
    pre = s * (1.0 / R) + jnp.mean(b1_ref[...], axis=0, keepdims=True)
    logits = jnp.dot(pre, wl_ref[...], preferred_element_type=jnp.float32)
    logits = logits + bl_ref[...]
    m = jnp.max(logits, axis=1, keepdims=True)
    lse = jnp.log(jnp.sum(jnp.exp(logits - m), axis=1, keepdims=True)) + m
    out_ref[...] = logits - lse


def _tc_head(A1, cnt, b1, Wl, blf):
    return pl.pallas_call(
        _tc_head_body,
        grid=(N // _BN,),
        in_specs=[
            pl.BlockSpec((R, _BN, W), lambda i: (0, i, 0)),
            pl.BlockSpec((R, _BN, W), lambda i: (0, i, 0)),
            pl.BlockSpec((R, H), lambda i: (0, 0)),
            pl.BlockSpec((H, OUT), lambda i: (0, 0)),
            pl.BlockSpec((1, OUT), lambda i: (0, 0)),
        ],
        out_specs=pl.BlockSpec((_BN, OUT), lambda i: (i, 0)),
        out_shape=jax.ShapeDtypeStruct((N, OUT), jnp.float32),
    )(A1, cnt, b1, Wl, blf)


def _sc_cnt_body(didx_hbm, z_hbm, o_hbm, out_hbm, acc, gdst, buf0, buf1):
    c = lax.axis_index("c")
    s = lax.axis_index("s")
    pltpu.sync_copy(o_hbm, buf1)

    for r_loc in range(R // NC):
        r = c * (R // NC) + r_loc
        pltpu.sync_copy(didx_hbm.at[r, s], gdst)

        pltpu.sync_copy(z_hbm, buf0)
        for k in range(5):
            row0 = s * TROWS + k * CH
            nrows = min(CH, TROWS - k * CH)
            pltpu.sync_copy(buf0.at[pl.ds(0, nrows)],
                            acc.at[pl.ds(row0, nrows)])
        @pl.when(s == NS - 1)
        def _():
            pltpu.sync_copy(buf0.at[pl.ds(0, TAIL + 8)],
                            acc.at[pl.ds(NS * TROWS, TAIL + 8)])

        plsc.subcore_barrier()

        @pl.loop(0, NCHUNK)
        def _scatter(j):
            pltpu.sync_copy(buf1, acc.at[gdst.at[j]], add=True)

        plsc.subcore_barrier()

        for k in range(5):
            row0 = s * TROWS + k * CH
            nrows = min(CH, TROWS - k * CH)
            pltpu.sync_copy(acc.at[pl.ds(row0, nrows)],
                            out_hbm.at[pl.ds(r * N + row0, nrows)])
        @pl.when(s == NS - 1)
        def _():
            pltpu.sync_copy(acc.at[pl.ds(NS * TROWS, TAIL)],
                            out_hbm.at[pl.ds(r * N + NS * TROWS, TAIL)])

        plsc.subcore_barrier()


def _sc_cnt(didx):
    mesh = plsc.VectorSubcoreMesh(core_axis_name="c", subcore_axis_name="s",
                                  num_cores=NC, num_subcores=NS)
    kern = pl.kernel(
        _sc_cnt_body,
        out_type=jax.ShapeDtypeStruct((R * N, W), jnp.float32),
        mesh=mesh,
        scratch_types=[
            pltpu.VMEM_SHARED((ACC_ROWS, W), jnp.float32),
            pltpu.VMEM((NCHUNK, CH), jnp.int32),
            pltpu.VMEM((CH, W), jnp.float32),
            pltpu.VMEM((CH, W), jnp.float32),
        ],
    )
    return kern(didx, jnp.zeros((CH, W), jnp.float32),
                jnp.ones((CH, W), jnp.float32))


def _sc_agg_body(t_hbm, gidx_hbm, didx_hbm, z_hbm, out_hbm,
                 acc, gsrc, gdst, buf0, buf1, sem0, sem1):
    c = lax.axis_index("c")
    s = lax.axis_index("s")

    for r_loc in range(R // NC):
        r = c * (R // NC) + r_loc

        pltpu.sync_copy(gidx_hbm.at[r, s], gsrc)
        pltpu.sync_copy(didx_hbm.at[r, s], gdst)

        pltpu.sync_copy(z_hbm, buf0)
        for k in range(5):
            row0 = s * TROWS + k * CH
            nrows = min(CH, TROWS - k * CH)
            pltpu.sync_copy(buf0.at[pl.ds(0, nrows)],
                            acc.at[pl.ds(row0, nrows)])
        @pl.when(s == NS - 1)
        def _():
            pltpu.sync_copy(buf0.at[pl.ds(0, TAIL + 8)],
                            acc.at[pl.ds(NS * TROWS, TAIL + 8)])

        pltpu.async_copy(t_hbm.at[gsrc.at[0]], buf0, sem0)
        pltpu.async_copy(t_hbm.at[gsrc.at[1]], buf1, sem1)

        plsc.subcore_barrier()

        @pl.loop(0, NCHUNK - 2, step=2)
        def _scatter(j):
            for b, (buf, sem) in enumerate(((buf0, sem0), (buf1, sem1))):
                jj = j + b
                pltpu.make_async_copy(t_hbm.at[gsrc.at[jj]], buf, sem).wait()
                pltpu.sync_copy(buf, acc.at[gdst.at[jj]], add=True)
                pltpu.async_copy(t_hbm.at[gsrc.at[jj + 2]], buf, sem)

        for b, (buf, sem) in enumerate(((buf0, sem0), (buf1, sem1))):
            jj = NCHUNK - 2 + b
            pltpu.make_async_copy(t_hbm.at[gsrc.at[jj]], buf, sem).wait()
            pltpu.sync_copy(buf, acc.at[gdst.at[jj]], add=True)

        plsc.subcore_barrier()

        for k in range(5):
            row0 = s * TROWS + k * CH
            nrows = min(CH, TROWS - k * CH)
            pltpu.sync_copy(acc.at[pl.ds(row0, nrows)],
                            out_hbm.at[pl.ds(r * N + row0, nrows)])
        @pl.when(s == NS - 1)
        def _():
            pltpu.sync_copy(acc.at[pl.ds(NS * TROWS, TAIL)],
                            out_hbm.at[pl.ds(r * N + NS * TROWS, TAIL)])

        plsc.subcore_barrier()


def _sc_agg(T, gidx, didx):
    mesh = plsc.VectorSubcoreMesh(core_axis_name="c", subcore_axis_name="s",
                                  num_cores=NC, num_subcores=NS)
    scratch = [
        pltpu.VMEM_SHARED((ACC_ROWS, W), jnp.float32),
        pltpu.VMEM((NCHUNK, CH), jnp.int32),
        pltpu.VMEM((NCHUNK, CH), jnp.int32),
        pltpu.VMEM((CH, W), jnp.float32),
        pltpu.VMEM((CH, W), jnp.float32),
        pltpu.SemaphoreType.DMA,
        pltpu.SemaphoreType.DMA,
    ]
    kern = pl.kernel(
        _sc_agg_body,
        out_type=jax.ShapeDtypeStruct((R * N, W), jnp.float32),
        mesh=mesh,
        scratch_types=scratch,
    )
    return kern(T, gidx, didx, jnp.zeros((CH, W), jnp.float32))


def kernel(x, edge_index, W0, b0, W1, b1, Wl, bl):
    src = edge_index[0]
    dst = edge_index[1]

    def _prep(ix, pad_val):
        a = ix.reshape(EPR, R).T.reshape(R, NS, NCHUNK, CPC)
        return jnp.pad(a, ((0, 0), (0, 0), (0, 0), (0, CH - CPC)),
                       constant_values=pad_val)

    gidx0 = _prep(src, 0)
    didx_p = _prep(dst, N)
    pair_off = (jnp.arange(R, dtype=jnp.int32) // 2 * N).reshape(R, 1, 1, 1)
    gidx1 = gidx0 + pair_off

    W1p = W1.reshape(R // 2, 2, R * H, H).transpose(0, 2, 1, 3)
    W1p = W1p.reshape(R // 2, R * H, 2 * H)

    cnt = _sc_cnt(didx_p).reshape(R, N, W)
    A0 = _sc_agg(x, gidx0, didx_p)
    U = _tc_mid(A0.reshape(R, N, D), cnt, W0, b0.reshape(1, R * H), W1p)
    A1 = _sc_agg(U.reshape(R // 2 * N, 2 * H), gidx1, didx_p)
    return _tc_head(A1.reshape(R, N, W), cnt, b1, Wl, bl.reshape(1, OUT))


# speedup vs baseline: 1.1004x; 1.0593x over previous
import functools

import jax
import jax.numpy as jnp
from jax import lax
from jax.experimental import pallas as pl
from jax.experimental.pallas import tpu as pltpu
from jax.experimental.pallas import tpu_sc as plsc

N = 10000
E = 320000
D = 128
H = 64
R = 8
OUT = H // 2

NS = 16
NC = 2
EPR = E // R
EPT = EPR // NS
CH = 128
NCHUNK = EPT // 125
CPC = EPT // NCHUNK
TROWS = 624
TAIL = N - NS * TROWS
ACC_ROWS = N + 8
W = 128


_BN = 1000


def _tc_mid_body(a_ref, c_ref, w0_ref, b0_ref, w1_ref, out_ref):
    hs = []
    for r in range(R):
        z = jnp.dot(a_ref[r], w0_ref[r], preferred_element_type=jnp.float32)
        g = 32 * (r % (R // NC))
        recip = 1.0 / jnp.maximum(c_ref[r // (R // NC)][:, g:g + 1], 1.0)
        hs.append(z * recip)
    h = jnp.concatenate(hs, axis=1) + b0_ref[...]
    h = jnp.maximum(h, 0.0)
    for p in range(R // 2):
        out_ref[p] = jnp.dot(h, w1_ref[p], preferred_element_type=jnp.float32)


def _tc_mid(A0, cnt, W0, b0f, W1p):
    return pl.pallas_call(
        _tc_mid_body,
        grid=(N // _BN,),
        in_specs=[
            pl.BlockSpec((R, _BN, D), lambda i: (0, i, 0)),
            pl.BlockSpec((NC, _BN, W), lambda i: (0, i, 0)),
            pl.BlockSpec((R, D, H), lambda i: (0, 0, 0)),
            pl.BlockSpec((1, R * H), lambda i: (0, 0)),
            pl.BlockSpec((R // 2, R * H, 2 * H), lambda i: (0, 0, 0)),
        ],
        out_specs=pl.BlockSpec((R // 2, _BN, 2 * H), lambda i: (0, i, 0)),
        out_shape=jax.ShapeDtypeStruct((R // 2, N, 2 * H), jnp.float32),
    )(A0, cnt, W0, b0f, W1p)


def _tc_head_body(a_ref, c_ref, b1_ref, wl_ref, bl_ref, out_ref):
    s = None
    for r in range(R):
        off = (r % 2) * H
        g = 32 * (r % (R // NC))
        recip = 1.0 / jnp.maximum(c_ref[r // (R // NC)][:, g:g + 1], 1.0)
        t = a_ref[r][:, off:off + H] * recip
        s = t if s is None else s + t
    pre = s * (1.0 / R) + jnp.mean(b1_ref[...], axis=0, keepdims=True)
    logits = jnp.dot(pre, wl_ref[...], preferred_element_type=jnp.float32)
    logits = logits + bl_ref[...]
    m = jnp.max(logits, axis=1, keepdims=True)
    lse = jnp.log(jnp.sum(jnp.exp(logits - m), axis=1, keepdims=True)) + m
    out_ref[...] = logits - lse


def _tc_head(A1, cnt, b1, Wl, blf):
    return pl.pallas_call(
        _tc_head_body,
        grid=(N // _BN,),
        in_specs=[
            pl.BlockSpec((R, _BN, W), lambda i: (0, i, 0)),
            pl.BlockSpec((NC, _BN, W), lambda i: (0, i, 0)),
            pl.BlockSpec((R, H), lambda i: (0, 0)),
            pl.BlockSpec((H, OUT), lambda i: (0, 0)),
            pl.BlockSpec((1, OUT), lambda i: (0, 0)),
        ],
        out_specs=pl.BlockSpec((_BN, OUT), lambda i: (i, 0)),
        out_shape=jax.ShapeDtypeStruct((N, OUT), jnp.float32),
    )(A1, cnt, b1, Wl, blf)


def _sc_agg_body(with_cnt, *refs):
    if with_cnt:
        (t_hbm, gidx_hbm, didx_hbm, z_hbm, o_hbm, out_hbm, cnt_hbm,
         acc, gsrc, gdst, buf0, buf1, sem0, sem1) = refs
    else:
        (t_hbm, gidx_hbm, didx_hbm, z_hbm, out_hbm,
         acc, gsrc, gdst, buf0, buf1, sem0, sem1) = refs
    c = lax.axis_index("c")
    s = lax.axis_index("s")

    for r_loc in range(R // NC):
        r = c * (R // NC) + r_loc

        pltpu.sync_copy(gidx_hbm.at[r, s], gsrc)
        pltpu.sync_copy(didx_hbm.at[r, s], gdst)

        pltpu.sync_copy(z_hbm, buf0)
        for k in range(5):
            row0 = s * TROWS + k * CH
            nrows = min(CH, TROWS - k * CH)
            pltpu.sync_copy(buf0.at[pl.ds(0, nrows)],
                            acc.at[pl.ds(row0, nrows)])
        @pl.when(s == NS - 1)
        def _():
            pltpu.sync_copy(buf0.at[pl.ds(0, TAIL + 8)],
                            acc.at[pl.ds(NS * TROWS, TAIL + 8)])

        pltpu.async_copy(t_hbm.at[gsrc.at[0]], buf0, sem0)
        pltpu.async_copy(t_hbm.at[gsrc.at[1]], buf1, sem1)

        plsc.subcore_barrier()

        @pl.loop(0, NCHUNK - 2, step=2)
        def _scatter(j):
            for b, (buf, sem) in enumerate(((buf0, sem0), (buf1, sem1))):
                jj = j + b
                pltpu.make_async_copy(t_hbm.at[gsrc.at[jj]], buf, sem).wait()
                pltpu.sync_copy(buf, acc.at[gdst.at[jj]], add=True)
                pltpu.async_copy(t_hbm.at[gsrc.at[jj + 2]], buf, sem)

        for b, (buf, sem) in enumerate(((buf0, sem0), (buf1, sem1))):
            jj = NCHUNK - 2 + b
            pltpu.make_async_copy(t_hbm.at[gsrc.at[jj]], buf, sem).wait()
            pltpu.sync_copy(buf, acc.at[gdst.at[jj]], add=True)

        plsc.subcore_barrier()

        for k in range(5):
            row0 = s * TROWS + k * CH
            nrows = min(CH, TROWS - k * CH)
            pltpu.sync_copy(acc.at[pl.ds(row0, nrows)],
                            out_hbm.at[pl.ds(r * N + row0, nrows)])
        @pl.when(s == NS - 1)
        def _():
            pltpu.sync_copy(acc.at[pl.ds(NS * TROWS, TAIL)],
                            out_hbm.at[pl.ds(r * N + NS * TROWS, TAIL)])

        plsc.subcore_barrier()

    if with_cnt:
        pltpu.sync_copy(z_hbm, buf0)
        for k in range(5):
            row0 = s * TROWS + k * CH
            nrows = min(CH, TROWS - k * CH)
            pltpu.sync_copy(buf0.at[pl.ds(0, nrows)],
                            acc.at[pl.ds(row0, nrows)])
        @pl.when(s == NS - 1)
        def _():
            pltpu.sync_copy(buf0.at[pl.ds(0, TAIL + 8)],
                            acc.at[pl.ds(NS * TROWS, TAIL + 8)])

        plsc.subcore_barrier()

        for r_loc in range(R // NC):
            r = c * (R // NC) + r_loc
            pltpu.sync_copy(didx_hbm.at[r, s], gdst)
            pltpu.sync_copy(o_hbm.at[r_loc], buf1)

            @pl.loop(0, NCHUNK)
            def _scatter_cnt(j):
                pltpu.sync_copy(buf1, acc.at[gdst.at[j]], add=True)

        plsc.subcore_barrier()

        for k in range(5):
            row0 = s * TROWS + k * CH
            nrows = min(CH, TROWS - k * CH)
            pltpu.sync_copy(acc.at[pl.ds(row0, nrows)],
                            cnt_hbm.at[pl.ds(c * N + row0, nrows)])
        @pl.when(s == NS - 1)
        def _():
            pltpu.sync_copy(acc.at[pl.ds(NS * TROWS, TAIL)],
                            cnt_hbm.at[pl.ds(c * N + NS * TROWS, TAIL)])


def _sc_agg(T, gidx, didx, with_cnt=False):
    mesh = plsc.VectorSubcoreMesh(core_axis_name="c", subcore_axis_name="s",
                                  num_cores=NC, num_subcores=NS)
    scratch = [
        pltpu.VMEM_SHARED((ACC_ROWS, W), jnp.float32),
        pltpu.VMEM((NCHUNK, CH), jnp.int32),
        pltpu.VMEM((NCHUNK, CH), jnp.int32),
        pltpu.VMEM((CH, W), jnp.float32),
        pltpu.VMEM((CH, W), jnp.float32),
        pltpu.SemaphoreType.DMA,
        pltpu.SemaphoreType.DMA,
    ]
    if with_cnt:
        out_type = (jax.ShapeDtypeStruct((R * N, W), jnp.float32),
                    jax.ShapeDtypeStruct((NC * N, W), jnp.float32))
    else:
        out_type = jax.ShapeDtypeStruct((R * N, W), jnp.float32)
    kern = pl.kernel(
        functools.partial(_sc_agg_body, with_cnt),
        out_type=out_type,
        mesh=mesh,
        scratch_types=scratch,
    )
    z = jnp.zeros((CH, W), jnp.float32)
    if with_cnt:
        lane_grp = jnp.arange(W, dtype=jnp.int32) // 32
        ones_pat = (lane_grp[None, None, :] ==
                    jnp.arange(R // NC, dtype=jnp.int32)[:, None, None])
        ones_pat = jnp.broadcast_to(ones_pat, (R // NC, CH, W))
        return kern(T, gidx, didx, z, ones_pat.astype(jnp.float32))
    return kern(T, gidx, didx, z)


def kernel(x, edge_index, W0, b0, W1, b1, Wl, bl):
    src = edge_index[0]
    dst = edge_index[1]

    def _prep(ix, pad_val):
        a = ix.reshape(EPR, R).T.reshape(R, NS, NCHUNK, CPC)
        return jnp.pad(a, ((0, 0), (0, 0), (0, 0), (0, CH - CPC)),
                       constant_values=pad_val)

    gidx0 = _prep(src, 0)
    didx_p = _prep(dst, N)
    pair_off = (jnp.arange(R, dtype=jnp.int32) // 2 * N).reshape(R, 1, 1, 1)
    gidx1 = gidx0 + pair_off

    W1p = W1.reshape(R // 2, 2, R * H, H).transpose(0, 2, 1, 3)
    W1p = W1p.reshape(R // 2, R * H, 2 * H)

    A0, cntp = _sc_agg(x, gidx0, didx_p, with_cnt=True)
    cnt = cntp.reshape(NC, N, W)
    U = _tc_mid(A0.reshape(R, N, D), cnt, W0, b0.reshape(1, R * H), W1p)
    A1 = _sc_agg(U.reshape(R // 2 * N, 2 * H), gidx1, didx_p)
    return _tc_head(A1.reshape(R, N, W), cnt, b1, Wl, bl.reshape(1, OUT))
